# baseline (speedup 1.0000x reference)
import functools

import jax
import jax.numpy as jnp
from jax import lax
from jax.experimental import pallas as pl
from jax.experimental.pallas import tpu as pltpu
from jax.experimental.pallas import tpu_sc as plsc

NUM_ROWS = 1024
SEQ = 200
DIM = 128

_info = plsc.get_sparse_core_info()
NC, NS = _info.num_cores, _info.num_subcores
NW = NC * NS
B = NUM_ROWS * SEQ
B_PER_W = B // NW
CHUNK = 128
NCHUNK = B_PER_W // CHUNK
PAIR = 2 * CHUNK
NPAIR = NCHUNK // 2
NBUF = 3

_mesh = plsc.VectorSubcoreMesh(core_axis_name="c", subcore_axis_name="s")


@functools.partial(
    pl.kernel,
    mesh=_mesh,
    out_type=jax.ShapeDtypeStruct((B, DIM), jnp.float32),
    scratch_types=[
        pltpu.VMEM((NCHUNK, CHUNK), jnp.int32),
        pltpu.VMEM((NBUF, PAIR, DIM), jnp.float32),
    ]
    + [pltpu.SemaphoreType.DMA] * (2 * NBUF),
)
def _emb_gather(idx_hbm, table_hbm, out_hbm, idx_v, rows_v, *sems):
    gsem, wsem = sems[:NBUF], sems[NBUF:]
    wid = lax.axis_index("s") * NC + lax.axis_index("c")
    base = wid * B_PER_W
    pltpu.sync_copy(idx_hbm.at[wid], idx_v)

    def start_pair_gather(p, b):
        for h in range(2):
            pltpu.async_copy(
                table_hbm.at[idx_v.at[2 * p + h]],
                rows_v.at[b, pl.ds(h * CHUNK, CHUNK)],
                gsem[b],
            )

    def wait_pair_gather(b):
        pltpu.make_async_copy(
            table_hbm.at[pl.ds(0, PAIR)], rows_v.at[b], gsem[b]
        ).wait()

    def start_write(p, b):
        pltpu.async_copy(
            rows_v.at[b], out_hbm.at[pl.ds(base + p * PAIR, PAIR)], wsem[b]
        )

    def wait_write(b):
        pltpu.make_async_copy(
            rows_v.at[b], out_hbm.at[pl.ds(base, PAIR)], wsem[b]
        ).wait()

    start_pair_gather(0, 0)
    start_pair_gather(1, 1)

    NITER = (NPAIR - 1) // NBUF

    def outer(i, carry):
        for b in range(NBUF):
            p = i * NBUF + b
            bn = (b + 2) % NBUF
            wait_pair_gather(b)
            start_write(p, b)
            if b == 0:
                @pl.when(i >= 1)
                def _():
                    wait_write(bn)

                start_pair_gather(p + 2, bn)
            elif b == 1:
                wait_write(bn)
                start_pair_gather(p + 2, bn)
            else:
                wait_write(bn)

                @pl.when(i < NITER - 1)
                def _():
                    start_pair_gather(p + 2, bn)
        return carry

    lax.fori_loop(0, NITER, outer, 0)
    wait_pair_gather(0)
    start_write(NPAIR - 1, 0)
    wait_write((NPAIR - 2) % NBUF)
    wait_write((NPAIR - 1) % NBUF)


def kernel(input_, weight):
    idx = input_.astype(jnp.int32).reshape(NW, NCHUNK, CHUNK)
    out = _emb_gather(idx, weight)
    return out.reshape(NUM_ROWS, SEQ, DIM)


# speedup vs baseline: 1.0141x; 1.0141x over previous
import functools

import jax
import jax.numpy as jnp
from jax import lax
from jax.experimental import pallas as pl
from jax.experimental.pallas import tpu as pltpu
from jax.experimental.pallas import tpu_sc as plsc

NUM_ROWS = 1024
SEQ = 200
DIM = 128

_info = plsc.get_sparse_core_info()
NC, NS = _info.num_cores, _info.num_subcores
NW = NC * NS
B = NUM_ROWS * SEQ
B_PER_W = B // NW
CHUNK = 128
NCHUNK = B_PER_W // CHUNK
NBUF = 5
LOOK = 3
NITER = NCHUNK // NBUF

_mesh = plsc.VectorSubcoreMesh(core_axis_name="c", subcore_axis_name="s")


@functools.partial(
    pl.kernel,
    mesh=_mesh,
    out_type=jax.ShapeDtypeStruct((B, DIM), jnp.float32),
    scratch_types=[
        pltpu.VMEM((NCHUNK, CHUNK), jnp.int32),
        pltpu.VMEM((NBUF, CHUNK, DIM), jnp.float32),
    ]
    + [pltpu.SemaphoreType.DMA] * (2 * NBUF),
    compiler_params=pltpu.CompilerParams(
        disable_bounds_checks=True,
        disable_semaphore_checks=True,
        skip_device_barrier=True,
    ),
)
def _emb_gather(idx_hbm, table_hbm, out_hbm, idx_v, rows_v, *sems):
    gsem, wsem = sems[:NBUF], sems[NBUF:]
    wid = lax.axis_index("s") * NC + lax.axis_index("c")
    base = wid * B_PER_W
    pltpu.sync_copy(idx_hbm.at[wid], idx_v)

    def start_gather(j, b):
        pltpu.async_copy(table_hbm.at[idx_v.at[j]], rows_v.at[b], gsem[b])

    def wait_gather(b):
        pltpu.make_async_copy(
            table_hbm.at[pl.ds(0, CHUNK)], rows_v.at[b], gsem[b]
        ).wait()

    def start_write(j, b):
        pltpu.async_copy(
            rows_v.at[b], out_hbm.at[pl.ds(base + j * CHUNK, CHUNK)], wsem[b]
        )

    def wait_write(b):
        pltpu.make_async_copy(
            rows_v.at[b], out_hbm.at[pl.ds(base, CHUNK)], wsem[b]
        ).wait()

    for p in range(LOOK):
        start_gather(p, p)

    def outer(i, carry):
        for b in range(NBUF):
            j = i * NBUF + b
            wait_gather(b)
            start_write(j, b)
            bn = (b + LOOK) % NBUF
            if b < NBUF - LOOK:
                @pl.when(i >= 1)
                def _():
                    wait_write(bn)

                start_gather(j + LOOK, bn)
            else:
                wait_write(bn)

                @pl.when(i < NITER - 1)
                def _():
                    start_gather(j + LOOK, bn)
        return carry

    lax.fori_loop(0, NITER, outer, 0)
    for b in range(LOOK, NBUF):
        wait_write(b)


def kernel(input_, weight):
    idx = input_.astype(jnp.int32).reshape(NW, NCHUNK, CHUNK)
    out = _emb_gather(idx, weight)
    return out.reshape(NUM_ROWS, SEQ, DIM)


# speedup vs baseline: 1.0196x; 1.0054x over previous
import functools

import jax
import jax.numpy as jnp
from jax import lax
from jax.experimental import pallas as pl
from jax.experimental.pallas import tpu as pltpu
from jax.experimental.pallas import tpu_sc as plsc

NUM_ROWS = 1024
SEQ = 200
DIM = 128

_info = plsc.get_sparse_core_info()
NC, NS = _info.num_cores, _info.num_subcores
NW = NC * NS
B = NUM_ROWS * SEQ
B_PER_W = B // NW
CHUNK = 128
NCHUNK = B_PER_W // CHUNK
NBUF = 5
LOOK = 3
NITER = NCHUNK // NBUF

_mesh = plsc.VectorSubcoreMesh(core_axis_name="c", subcore_axis_name="s")


@functools.partial(
    pl.kernel,
    mesh=_mesh,
    out_type=jax.ShapeDtypeStruct((B, DIM), jnp.float32),
    scratch_types=[
        pltpu.VMEM((NCHUNK, CHUNK), jnp.int32),
        pltpu.VMEM((NBUF, CHUNK, DIM), jnp.float32),
    ]
    + [pltpu.SemaphoreType.DMA] * (2 * NBUF),
)
def _emb_gather(idx_hbm, table_hbm, out_hbm, idx_v, rows_v, *sems):
    gsem, wsem = sems[:NBUF], sems[NBUF:]
    wid = lax.axis_index("s") * NC + lax.axis_index("c")
    base = wid * B_PER_W
    pltpu.sync_copy(idx_hbm.at[wid], idx_v)

    def start_gather(j, b):
        pltpu.async_copy(table_hbm.at[idx_v.at[j]], rows_v.at[b], gsem[b])

    def wait_gather(b):
        pltpu.make_async_copy(
            table_hbm.at[pl.ds(0, CHUNK)], rows_v.at[b], gsem[b]
        ).wait()

    def start_write(j, b):
        pltpu.async_copy(
            rows_v.at[b], out_hbm.at[pl.ds(base + j * CHUNK, CHUNK)], wsem[b]
        )

    def wait_write(b):
        pltpu.make_async_copy(
            rows_v.at[b], out_hbm.at[pl.ds(base, CHUNK)], wsem[b]
        ).wait()

    for p in range(LOOK):
        start_gather(p, p)

    def outer(i, carry):
        for b in range(NBUF):
            j = i * NBUF + b
            wait_gather(b)
            start_write(j, b)
            bn = (b + LOOK) % NBUF
            if b < NBUF - LOOK:
                @pl.when(i >= 1)
                def _():
                    wait_write(bn)

                start_gather(j + LOOK, bn)
            else:
                wait_write(bn)

                @pl.when(i < NITER - 1)
                def _():
                    start_gather(j + LOOK, bn)
        return carry

    lax.fori_loop(0, NITER, outer, 0)
    for b in range(LOOK, NBUF):
        wait_write(b)


def kernel(input_, weight):
    idx = input_.astype(jnp.int32).reshape(NW, NCHUNK, CHUNK)
    out = _emb_gather(idx, weight)
    return out.reshape(NUM_ROWS, SEQ, DIM)
